# Initial kernel scaffold; baseline (speedup 1.0000x reference)
import jax
import jax.numpy as jnp
from jax.experimental import pallas as pl


def kernel(x, edge_index, W1, b1, g1, be1, W2, b2, g2, be2, Wfc, bfc):
    raise NotImplementedError("write your pallas kernel here")


# speedup vs baseline: 7.7514x; 7.7514x over previous
import jax
import jax.numpy as jnp
from jax import lax
from jax.experimental import pallas as pl
from jax.experimental.pallas import tpu as pltpu
from jax.experimental.pallas import tpu_sc as plsc

N = 10000
E = 320000
D = 128
DH = D // 2
D_OUT = 40

NC = 2
NS = 16
EPT = E // NS
K = 80
NB = EPT // K
NA = 10240
NPT = NA // NS
DEG_W = 16
EPW = E // (NC * NS)

BM = 400


def _deg_body(edge, outp, ones_v, idx_v, zbuf, acc, sem):
    c = lax.axis_index("c")
    s = lax.axis_index("s")
    wid = c * NS + s

    def fill(i, carry):
        ones_v[i, :] = jnp.ones((DEG_W,), jnp.float32)
        zbuf[i, :] = jnp.zeros((DEG_W,), jnp.float32)
        return carry

    lax.fori_loop(0, NPT, fill, 0)
    pltpu.sync_copy(zbuf, acc.at[pl.ds(s * NPT, NPT)])
    plsc.subcore_barrier()

    base = wid * EPW

    def step(b, carry):
        pltpu.sync_copy(edge.at[pl.ds(E + base + b * K, K)], idx_v)
        pltpu.sync_copy(ones_v.at[pl.ds(0, K)], acc.at[idx_v], add=True)
        return carry

    lax.fori_loop(0, NB // NC, step, 0)
    plsc.subcore_barrier()
    pltpu.sync_copy(acc.at[pl.ds(s * NPT, NPT)], outp.at[c, pl.ds(s * NPT, NPT)])


def _deg_call(edge_ravel):
    mesh = plsc.VectorSubcoreMesh(core_axis_name="c", subcore_axis_name="s")
    f = pl.kernel(
        _deg_body,
        compiler_params=pltpu.CompilerParams(use_tc_tiling_on_sc=False),
        out_type=jax.ShapeDtypeStruct((NC, NA, DEG_W), jnp.float32),
        mesh=mesh,
        scratch_types=[
            pltpu.VMEM((NPT, DEG_W), jnp.float32),
            pltpu.VMEM((K,), jnp.int32),
            pltpu.VMEM((NPT, DEG_W), jnp.float32),
            pltpu.VMEM_SHARED((NA, DEG_W), jnp.float32),
            pltpu.SemaphoreType.DMA,
        ],
    )
    return f(edge_ravel)


def _agg_body(hp, edge, outp, src_v, dst_v, rows, zbuf, acc, sem):
    c = lax.axis_index("c")
    s = lax.axis_index("s")

    ZR = 128

    def zf(i, carry):
        for j in range(DH // 16):
            zbuf[i, pl.ds(j * 16, 16)] = jnp.zeros((16,), jnp.float32)
        return carry

    lax.fori_loop(0, ZR, zf, 0)
    for j in range(NPT // ZR):
        pltpu.sync_copy(zbuf, acc.at[pl.ds(s * NPT + j * ZR, ZR)])
    plsc.subcore_barrier()

    base = s * EPT
    table = hp.at[c]

    def step(b, carry):
        off = base + b * K
        pltpu.sync_copy(edge.at[pl.ds(off, K)], src_v)
        pltpu.sync_copy(edge.at[pl.ds(E + off, K)], dst_v)
        pltpu.async_copy(table.at[src_v], rows, sem).wait()
        pltpu.sync_copy(rows, acc.at[dst_v], add=True)
        return carry

    lax.fori_loop(0, NB, step, 0)
    plsc.subcore_barrier()
    pltpu.sync_copy(acc.at[pl.ds(s * NPT, NPT)], outp.at[c, pl.ds(s * NPT, NPT)])


def _agg_call(hp, edge_ravel):
    mesh = plsc.VectorSubcoreMesh(core_axis_name="c", subcore_axis_name="s")
    f = pl.kernel(
        _agg_body,
        compiler_params=pltpu.CompilerParams(use_tc_tiling_on_sc=False),
        out_type=jax.ShapeDtypeStruct((NC, NA, DH), jnp.float32),
        mesh=mesh,
        scratch_types=[
            pltpu.VMEM((K,), jnp.int32),
            pltpu.VMEM((K,), jnp.int32),
            pltpu.VMEM((K, DH), jnp.float32),
            pltpu.VMEM((128, DH), jnp.float32),
            pltpu.VMEM_SHARED((NA, DH), jnp.float32),
            pltpu.SemaphoreType.DMA,
        ],
    )
    return f(hp, edge_ravel)


def _dinv(degp_ref):
    dd = degp_ref[0] + degp_ref[1]
    return lax.rsqrt(dd[:, 0:1] + 1.0)


def _pre_body(x_ref, w_ref, degp_ref, out_ref):
    h = jnp.dot(x_ref[...], w_ref[...], preferred_element_type=jnp.float32)
    h = h * _dinv(degp_ref)
    out_ref[0] = h[:, :DH]
    out_ref[1] = h[:, DH:]


def _mid_body(p_ref, hp_ref, degp_ref, w_ref, a_ref, bc_ref, x1_ref, hp2_ref):
    dinv = _dinv(degp_ref)
    agg = jnp.concatenate([p_ref[0] + hp_ref[0], p_ref[1] + hp_ref[1]], axis=1)
    x1 = jnp.maximum(agg * dinv * a_ref[...] + bc_ref[...], 0.0)
    x1_ref[...] = x1
    h2 = jnp.dot(x1, w_ref[...], preferred_element_type=jnp.float32) * dinv
    hp2_ref[0] = h2[:, :DH]
    hp2_ref[1] = h2[:, DH:]


def _fin_body(q_ref, hp2_ref, degp_ref, x0_ref, x1_ref, wfc_ref, a_ref, bc_ref,
              bfc_ref, out_ref):
    dinv = _dinv(degp_ref)
    agg = jnp.concatenate([q_ref[0] + hp2_ref[0], q_ref[1] + hp2_ref[1]], axis=1)
    x2 = jnp.maximum(agg * dinv * a_ref[...] + bc_ref[...], 0.0)
    acc = jnp.dot(x0_ref[...], wfc_ref[0:D], preferred_element_type=jnp.float32)
    acc += jnp.dot(x1_ref[...], wfc_ref[D:2 * D], preferred_element_type=jnp.float32)
    acc += jnp.dot(x2, wfc_ref[2 * D:3 * D], preferred_element_type=jnp.float32)
    out_ref[...] = acc + bfc_ref[...]


def _row_spec(width):
    return pl.BlockSpec((BM, width), lambda i: (i, 0))


def _full_spec(shape):
    nd = len(shape)
    return pl.BlockSpec(shape, lambda i: (0,) * nd)


def _half_spec():
    return pl.BlockSpec((NC, BM, DH), lambda i: (0, i, 0))


def _pre_call(x, w, degp):
    return pl.pallas_call(
        _pre_body,
        grid=(N // BM,),
        in_specs=[_row_spec(D), _full_spec((D, D)),
                  pl.BlockSpec((NC, BM, DEG_W), lambda i: (0, i, 0))],
        out_specs=_half_spec(),
        out_shape=jax.ShapeDtypeStruct((NC, N, DH), jnp.float32),
    )(x, w, degp)


def _mid_call(p, hp, degp, w, a, bc):
    return pl.pallas_call(
        _mid_body,
        grid=(N // BM,),
        in_specs=[_half_spec(), _half_spec(),
                  pl.BlockSpec((NC, BM, DEG_W), lambda i: (0, i, 0)),
                  _full_spec((D, D)), _full_spec((1, D)), _full_spec((1, D))],
        out_specs=[_row_spec(D), _half_spec()],
        out_shape=[jax.ShapeDtypeStruct((N, D), jnp.float32),
                   jax.ShapeDtypeStruct((NC, N, DH), jnp.float32)],
    )(p, hp, degp, w, a, bc)


def _fin_call(q, hp2, degp, x0, x1, wfc, a, bc, bfc):
    return pl.pallas_call(
        _fin_body,
        grid=(N // BM,),
        in_specs=[_half_spec(), _half_spec(),
                  pl.BlockSpec((NC, BM, DEG_W), lambda i: (0, i, 0)),
                  _row_spec(D), _row_spec(D), _full_spec((3 * D, D_OUT)),
                  _full_spec((1, D)), _full_spec((1, D)),
                  _full_spec((1, D_OUT))],
        out_specs=_row_spec(D_OUT),
        out_shape=jax.ShapeDtypeStruct((N, D_OUT), jnp.float32),
    )(q, hp2, degp, x0, x1, wfc, a, bc, bfc)


def kernel(x, edge_index, W1, b1, g1, be1, W2, b2, g2, be2, Wfc, bfc):
    sbn = 1.0 / jnp.sqrt(jnp.float32(1.0 + 1e-5))
    a1 = (g1 * sbn).reshape(1, D)
    bc1 = (b1 * g1 * sbn + be1).reshape(1, D)
    a2 = (g2 * sbn).reshape(1, D)
    bc2 = (b2 * g2 * sbn + be2).reshape(1, D)

    er = edge_index.reshape(2 * E)
    degp = _deg_call(er)
    h1p = _pre_call(x, W1, degp)
    p1 = _agg_call(h1p, er)
    x1, h2p = _mid_call(p1, h1p, degp, W2, a1, bc1)
    p2 = _agg_call(h2p, er)
    out = _fin_call(p2, h2p, degp, x, x1, Wfc, a2, bc2, bfc.reshape(1, D_OUT))
    return out


# speedup vs baseline: 14.2340x; 1.8363x over previous
import jax
import jax.numpy as jnp
from jax import lax
from jax.experimental import pallas as pl
from jax.experimental.pallas import tpu as pltpu
from jax.experimental.pallas import tpu_sc as plsc

N = 10000
E = 320000
D = 128
DH = D // 2
D_OUT = 40

NC = 2
NS = 16
EPT = E // NS
K = 80
NB = EPT // K
NA = 10240
NPT = NA // NS
DEG_W = 16
EPW = E // (NC * NS)

BM = 400


def _deg_body(edge, outp, ones_v, idx0, idx1, zbuf, acc, semi0, semi1):
    c = lax.axis_index("c")
    s = lax.axis_index("s")
    wid = c * NS + s

    def fill(i, carry):
        ones_v[i, :] = jnp.ones((DEG_W,), jnp.float32)
        zbuf[i, :] = jnp.zeros((DEG_W,), jnp.float32)
        return carry

    lax.fori_loop(0, NPT, fill, 0)
    pltpu.sync_copy(zbuf, acc.at[pl.ds(s * NPT, NPT)])
    plsc.subcore_barrier()

    base = wid * EPW
    KD = 40
    NBD = EPW // KD
    idxs = (idx0, idx1)
    semis = (semi0, semi1)

    def idx_start(b, j):
        pltpu.async_copy(edge.at[pl.ds(E + base + b * KD, KD)], idxs[j], semis[j])

    def idx_wait(j):
        pltpu.make_async_copy(edge.at[pl.ds(E + base, KD)], idxs[j], semis[j]).wait()

    idx_start(0, 0)

    def body(g, carry):
        for j in (0, 1):
            b = g * 2 + j

            @pl.when(b + 1 < NBD)
            def _():
                idx_start(b + 1, 1 - j)

            idx_wait(j)
            pltpu.sync_copy(ones_v.at[pl.ds(0, KD)], acc.at[idxs[j]], add=True)
        return carry

    lax.fori_loop(0, NBD // 2, body, 0)
    plsc.subcore_barrier()
    pltpu.sync_copy(acc.at[pl.ds(s * NPT, NPT)], outp.at[c, pl.ds(s * NPT, NPT)])


def _deg_call(edge_ravel):
    mesh = plsc.VectorSubcoreMesh(core_axis_name="c", subcore_axis_name="s")
    f = pl.kernel(
        _deg_body,
        compiler_params=pltpu.CompilerParams(use_tc_tiling_on_sc=False),
        out_type=jax.ShapeDtypeStruct((NC, NA, DEG_W), jnp.float32),
        mesh=mesh,
        scratch_types=[
            pltpu.VMEM((NPT, DEG_W), jnp.float32),
            pltpu.VMEM((40,), jnp.int32),
            pltpu.VMEM((40,), jnp.int32),
            pltpu.VMEM((NPT, DEG_W), jnp.float32),
            pltpu.VMEM_SHARED((NA, DEG_W), jnp.float32),
            pltpu.SemaphoreType.DMA,
            pltpu.SemaphoreType.DMA,
        ],
    )
    return f(edge_ravel)


def _agg_body(hp, edge, outp, src0, src1, dst0, dst1, rows0, rows1, zbuf, acc,
              semi0, semi1, semg0, semg1):
    c = lax.axis_index("c")
    s = lax.axis_index("s")

    ZR = 128

    def zf(i, carry):
        for j in range(DH // 16):
            zbuf[i, pl.ds(j * 16, 16)] = jnp.zeros((16,), jnp.float32)
        return carry

    lax.fori_loop(0, ZR, zf, 0)
    for j in range(NPT // ZR):
        pltpu.sync_copy(zbuf, acc.at[pl.ds(s * NPT + j * ZR, ZR)])
    plsc.subcore_barrier()

    base = s * EPT
    table = hp.at[c]
    srcs = (src0, src1)
    dsts = (dst0, dst1)
    rows = (rows0, rows1)
    semis = (semi0, semi1)
    semgs = (semg0, semg1)

    def idx_start(b, j):
        pltpu.async_copy(edge.at[pl.ds(base + b * K, K)], srcs[j], semis[j])
        pltpu.async_copy(edge.at[pl.ds(E + base + b * K, K)], dsts[j], semis[j])

    def idx_wait(j):
        pltpu.make_async_copy(edge.at[pl.ds(base, K)], srcs[j], semis[j]).wait()
        pltpu.make_async_copy(edge.at[pl.ds(base, K)], dsts[j], semis[j]).wait()

    def gather_start(j):
        pltpu.async_copy(table.at[srcs[j]], rows[j], semgs[j])

    def gather_wait(j):
        pltpu.make_async_copy(table.at[srcs[j]], rows[j], semgs[j]).wait()

    idx_start(0, 0)
    idx_wait(0)
    gather_start(0)
    idx_start(1, 1)

    def body(g, carry):
        for j in (0, 1):
            b = g * 2 + j
            gather_wait(j)

            @pl.when(b + 1 < NB)
            def _():
                idx_wait(1 - j)
                gather_start(1 - j)

            pltpu.sync_copy(rows[j], acc.at[dsts[j]], add=True)

            @pl.when(b + 2 < NB)
            def _():
                idx_start(b + 2, j)
        return carry

    lax.fori_loop(0, NB // 2, body, 0)
    plsc.subcore_barrier()
    pltpu.sync_copy(acc.at[pl.ds(s * NPT, NPT)], outp.at[c, pl.ds(s * NPT, NPT)])


def _agg_call(hp, edge_ravel):
    mesh = plsc.VectorSubcoreMesh(core_axis_name="c", subcore_axis_name="s")
    f = pl.kernel(
        _agg_body,
        compiler_params=pltpu.CompilerParams(use_tc_tiling_on_sc=False),
        out_type=jax.ShapeDtypeStruct((NC, NA, DH), jnp.float32),
        mesh=mesh,
        scratch_types=[
            pltpu.VMEM((K,), jnp.int32),
            pltpu.VMEM((K,), jnp.int32),
            pltpu.VMEM((K,), jnp.int32),
            pltpu.VMEM((K,), jnp.int32),
            pltpu.VMEM((K, DH), jnp.float32),
            pltpu.VMEM((K, DH), jnp.float32),
            pltpu.VMEM((128, DH), jnp.float32),
            pltpu.VMEM_SHARED((NA, DH), jnp.float32),
            pltpu.SemaphoreType.DMA,
            pltpu.SemaphoreType.DMA,
            pltpu.SemaphoreType.DMA,
            pltpu.SemaphoreType.DMA,
        ],
    )
    return f(hp, edge_ravel)


def _dinv(degp_ref):
    dd = degp_ref[0] + degp_ref[1]
    return lax.rsqrt(dd[:, 0:1] + 1.0)


def _pre_body(x_ref, w_ref, degp_ref, out_ref):
    h = jnp.dot(x_ref[...], w_ref[...], preferred_element_type=jnp.float32)
    h = h * _dinv(degp_ref)
    out_ref[0] = h[:, :DH]
    out_ref[1] = h[:, DH:]


def _mid_body(p_ref, hp_ref, degp_ref, w_ref, a_ref, bc_ref, x1_ref, hp2_ref):
    dinv = _dinv(degp_ref)
    agg = jnp.concatenate([p_ref[0] + hp_ref[0], p_ref[1] + hp_ref[1]], axis=1)
    x1 = jnp.maximum(agg * dinv * a_ref[...] + bc_ref[...], 0.0)
    x1_ref[...] = x1
    h2 = jnp.dot(x1, w_ref[...], preferred_element_type=jnp.float32) * dinv
    hp2_ref[0] = h2[:, :DH]
    hp2_ref[1] = h2[:, DH:]


def _fin_body(q_ref, hp2_ref, degp_ref, x0_ref, x1_ref, wfc_ref, a_ref, bc_ref,
              bfc_ref, out_ref):
    dinv = _dinv(degp_ref)
    agg = jnp.concatenate([q_ref[0] + hp2_ref[0], q_ref[1] + hp2_ref[1]], axis=1)
    x2 = jnp.maximum(agg * dinv * a_ref[...] + bc_ref[...], 0.0)
    acc = jnp.dot(x0_ref[...], wfc_ref[0:D], preferred_element_type=jnp.float32)
    acc += jnp.dot(x1_ref[...], wfc_ref[D:2 * D], preferred_element_type=jnp.float32)
    acc += jnp.dot(x2, wfc_ref[2 * D:3 * D], preferred_element_type=jnp.float32)
    out_ref[...] = acc + bfc_ref[...]


def _row_spec(width):
    return pl.BlockSpec((BM, width), lambda i: (i, 0))


def _full_spec(shape):
    nd = len(shape)
    return pl.BlockSpec(shape, lambda i: (0,) * nd)


def _half_spec():
    return pl.BlockSpec((NC, BM, DH), lambda i: (0, i, 0))


def _pre_call(x, w, degp):
    return pl.pallas_call(
        _pre_body,
        grid=(N // BM,),
        in_specs=[_row_spec(D), _full_spec((D, D)),
                  pl.BlockSpec((NC, BM, DEG_W), lambda i: (0, i, 0))],
        out_specs=_half_spec(),
        out_shape=jax.ShapeDtypeStruct((NC, N, DH), jnp.float32),
    )(x, w, degp)


def _mid_call(p, hp, degp, w, a, bc):
    return pl.pallas_call(
        _mid_body,
        grid=(N // BM,),
        in_specs=[_half_spec(), _half_spec(),
                  pl.BlockSpec((NC, BM, DEG_W), lambda i: (0, i, 0)),
                  _full_spec((D, D)), _full_spec((1, D)), _full_spec((1, D))],
        out_specs=[_row_spec(D), _half_spec()],
        out_shape=[jax.ShapeDtypeStruct((N, D), jnp.float32),
                   jax.ShapeDtypeStruct((NC, N, DH), jnp.float32)],
    )(p, hp, degp, w, a, bc)


def _fin_call(q, hp2, degp, x0, x1, wfc, a, bc, bfc):
    return pl.pallas_call(
        _fin_body,
        grid=(N // BM,),
        in_specs=[_half_spec(), _half_spec(),
                  pl.BlockSpec((NC, BM, DEG_W), lambda i: (0, i, 0)),
                  _row_spec(D), _row_spec(D), _full_spec((3 * D, D_OUT)),
                  _full_spec((1, D)), _full_spec((1, D)),
                  _full_spec((1, D_OUT))],
        out_specs=_row_spec(D_OUT),
        out_shape=jax.ShapeDtypeStruct((N, D_OUT), jnp.float32),
    )(q, hp2, degp, x0, x1, wfc, a, bc, bfc)


def kernel(x, edge_index, W1, b1, g1, be1, W2, b2, g2, be2, Wfc, bfc):
    sbn = 1.0 / jnp.sqrt(jnp.float32(1.0 + 1e-5))
    a1 = (g1 * sbn).reshape(1, D)
    bc1 = (b1 * g1 * sbn + be1).reshape(1, D)
    a2 = (g2 * sbn).reshape(1, D)
    bc2 = (b2 * g2 * sbn + be2).reshape(1, D)

    er = edge_index.reshape(2 * E)
    degp = _deg_call(er)
    h1p = _pre_call(x, W1, degp)
    p1 = _agg_call(h1p, er)
    x1, h2p = _mid_call(p1, h1p, degp, W2, a1, bc1)
    p2 = _agg_call(h2p, er)
    out = _fin_call(p2, h2p, degp, x, x1, Wfc, a2, bc2, bfc.reshape(1, D_OUT))
    return out


# speedup vs baseline: 19.4598x; 1.3671x over previous
import jax
import jax.numpy as jnp
from jax import lax
from jax.experimental import pallas as pl
from jax.experimental.pallas import tpu as pltpu
from jax.experimental.pallas import tpu_sc as plsc

N = 10000
E = 320000
D = 128
DH = D // 2
D_OUT = 40

NC = 2
NS = 16
EPT = E // NS
K = 80
NB = EPT // K
NA = 10240
NPT = NA // NS
DEG_W = 16
EPW = E // (NC * NS)

BM = 400


def _deg_body(edge, outp, ones_v, idx0, idx1, zbuf, acc, semi0, semi1):
    c = lax.axis_index("c")
    s = lax.axis_index("s")
    wid = c * NS + s

    def fill(i, carry):
        ones_v[i, :] = jnp.ones((DEG_W,), jnp.float32)
        zbuf[i, :] = jnp.zeros((DEG_W,), jnp.float32)
        return carry

    lax.fori_loop(0, NPT, fill, 0)
    pltpu.sync_copy(zbuf, acc.at[pl.ds(s * NPT, NPT)])
    plsc.subcore_barrier()

    base = wid * EPW
    KD = 40
    NBD = EPW // KD
    idxs = (idx0, idx1)
    semis = (semi0, semi1)

    def idx_start(b, j):
        pltpu.async_copy(edge.at[pl.ds(E + base + b * KD, KD)], idxs[j], semis[j])

    def idx_wait(j):
        pltpu.make_async_copy(edge.at[pl.ds(E + base, KD)], idxs[j], semis[j]).wait()

    idx_start(0, 0)

    def body(g, carry):
        for j in (0, 1):
            b = g * 2 + j

            @pl.when(b + 1 < NBD)
            def _():
                idx_start(b + 1, 1 - j)

            idx_wait(j)
            pltpu.sync_copy(ones_v.at[pl.ds(0, KD)], acc.at[idxs[j]], add=True)
        return carry

    lax.fori_loop(0, NBD // 2, body, 0)
    plsc.subcore_barrier()
    pltpu.sync_copy(acc.at[pl.ds(s * NPT, NPT)], outp.at[c, pl.ds(s * NPT, NPT)])


def _deg_call(edge_ravel):
    mesh = plsc.VectorSubcoreMesh(core_axis_name="c", subcore_axis_name="s")
    f = pl.kernel(
        _deg_body,
        compiler_params=pltpu.CompilerParams(use_tc_tiling_on_sc=False),
        out_type=jax.ShapeDtypeStruct((NC, NA, DEG_W), jnp.float32),
        mesh=mesh,
        scratch_types=[
            pltpu.VMEM((NPT, DEG_W), jnp.float32),
            pltpu.VMEM((40,), jnp.int32),
            pltpu.VMEM((40,), jnp.int32),
            pltpu.VMEM((NPT, DEG_W), jnp.float32),
            pltpu.VMEM_SHARED((NA, DEG_W), jnp.float32),
            pltpu.SemaphoreType.DMA,
            pltpu.SemaphoreType.DMA,
        ],
    )
    return f(edge_ravel)


def _agg_body(hp, edge, outp, src0, src1, src2, src3, dst0, dst1, dst2, dst3,
              rows0, rows1, rows2, rows3, zbuf, acc,
              sems0, sems1, sems2, sems3, semd0, semd1, semd2, semd3,
              semg0, semg1, semg2, semg3, semw0, semw1, semw2, semw3):
    c = lax.axis_index("c")
    s = lax.axis_index("s")

    ZR = 128

    def zf(i, carry):
        for j in range(DH // 16):
            zbuf[i, pl.ds(j * 16, 16)] = jnp.zeros((16,), jnp.float32)
        return carry

    lax.fori_loop(0, ZR, zf, 0)
    for j in range(NPT // ZR):
        pltpu.sync_copy(zbuf, acc.at[pl.ds(s * NPT + j * ZR, ZR)])
    plsc.subcore_barrier()

    base = s * EPT
    table = hp.at[c]
    srcs = (src0, src1, src2, src3)
    dsts = (dst0, dst1, dst2, dst3)
    rows = (rows0, rows1, rows2, rows3)
    semss = (sems0, sems1, sems2, sems3)
    semds = (semd0, semd1, semd2, semd3)
    semgs = (semg0, semg1, semg2, semg3)
    semws = (semw0, semw1, semw2, semw3)

    def src_start(b, j):
        pltpu.async_copy(edge.at[pl.ds(base + b * K, K)], srcs[j], semss[j])

    def src_wait(j):
        pltpu.make_async_copy(edge.at[pl.ds(base, K)], srcs[j], semss[j]).wait()

    def dst_start(b, j):
        pltpu.async_copy(edge.at[pl.ds(E + base + b * K, K)], dsts[j], semds[j])

    def dst_wait(j):
        pltpu.make_async_copy(edge.at[pl.ds(base, K)], dsts[j], semds[j]).wait()

    def gather_start(j):
        pltpu.async_copy(table.at[srcs[j]], rows[j], semgs[j])

    def gather_wait(j):
        pltpu.make_async_copy(table.at[srcs[j]], rows[j], semgs[j]).wait()

    def scatter_start(j):
        pltpu.async_copy(rows[j], acc.at[dsts[j]], semws[j], add=True)

    def scatter_wait(j):
        pltpu.make_async_copy(rows[j], acc.at[dsts[j]], semws[j]).wait()

    for j in range(4):
        src_start(j, j)
    dst_start(0, 0)
    dst_start(1, 1)
    src_wait(0)
    gather_start(0)
    src_wait(1)
    gather_start(1)

    NBP = NB + 2

    def body(g, carry):
        for j in range(4):
            b = g * 4 + j

            @pl.when(b < NB)
            def _():
                gather_wait(j)
                dst_wait(j)
                scatter_start(j)

            @pl.when(b + 2 < NB)
            def _():
                j2 = (j + 2) % 4

                @pl.when(b >= 2)
                def _():
                    scatter_wait(j2)

                dst_start(b + 2, j2)
                src_wait(j2)
                gather_start(j2)

            @pl.when(b + 4 < NB)
            def _():
                src_start(b + 4, j)

        return carry

    lax.fori_loop(0, NBP // 4, body, 0)
    for j in range(4):
        scatter_wait(j)
    plsc.subcore_barrier()
    pltpu.sync_copy(acc.at[pl.ds(s * NPT, NPT)], outp.at[c, pl.ds(s * NPT, NPT)])


def _agg_call(hp, edge_ravel):
    mesh = plsc.VectorSubcoreMesh(core_axis_name="c", subcore_axis_name="s")
    f = pl.kernel(
        _agg_body,
        compiler_params=pltpu.CompilerParams(use_tc_tiling_on_sc=False),
        out_type=jax.ShapeDtypeStruct((NC, NA, DH), jnp.float32),
        mesh=mesh,
        scratch_types=(
            [pltpu.VMEM((K,), jnp.int32) for _ in range(8)]
            + [pltpu.VMEM((K, DH), jnp.float32) for _ in range(4)]
            + [pltpu.VMEM((128, DH), jnp.float32),
               pltpu.VMEM_SHARED((NA, DH), jnp.float32)]
            + [pltpu.SemaphoreType.DMA for _ in range(16)]
        ),
    )
    return f(hp, edge_ravel)


def _dinv(degp_ref):
    dd = degp_ref[0] + degp_ref[1]
    return lax.rsqrt(dd[:, 0:1] + 1.0)


def _pre_body(x_ref, w_ref, degp_ref, out_ref):
    h = jnp.dot(x_ref[...], w_ref[...], preferred_element_type=jnp.float32)
    h = h * _dinv(degp_ref)
    out_ref[0] = h[:, :DH]
    out_ref[1] = h[:, DH:]


def _mid_body(p_ref, hp_ref, degp_ref, w_ref, a_ref, bc_ref, x1_ref, hp2_ref):
    dinv = _dinv(degp_ref)
    agg = jnp.concatenate([p_ref[0] + hp_ref[0], p_ref[1] + hp_ref[1]], axis=1)
    x1 = jnp.maximum(agg * dinv * a_ref[...] + bc_ref[...], 0.0)
    x1_ref[...] = x1
    h2 = jnp.dot(x1, w_ref[...], preferred_element_type=jnp.float32) * dinv
    hp2_ref[0] = h2[:, :DH]
    hp2_ref[1] = h2[:, DH:]


def _fin_body(q_ref, hp2_ref, degp_ref, x0_ref, x1_ref, wfc_ref, a_ref, bc_ref,
              bfc_ref, out_ref):
    dinv = _dinv(degp_ref)
    agg = jnp.concatenate([q_ref[0] + hp2_ref[0], q_ref[1] + hp2_ref[1]], axis=1)
    x2 = jnp.maximum(agg * dinv * a_ref[...] + bc_ref[...], 0.0)
    acc = jnp.dot(x0_ref[...], wfc_ref[0:D], preferred_element_type=jnp.float32)
    acc += jnp.dot(x1_ref[...], wfc_ref[D:2 * D], preferred_element_type=jnp.float32)
    acc += jnp.dot(x2, wfc_ref[2 * D:3 * D], preferred_element_type=jnp.float32)
    out_ref[...] = acc + bfc_ref[...]


def _row_spec(width):
    return pl.BlockSpec((BM, width), lambda i: (i, 0))


def _full_spec(shape):
    nd = len(shape)
    return pl.BlockSpec(shape, lambda i: (0,) * nd)


def _half_spec():
    return pl.BlockSpec((NC, BM, DH), lambda i: (0, i, 0))


def _pre_call(x, w, degp):
    return pl.pallas_call(
        _pre_body,
        grid=(N // BM,),
        in_specs=[_row_spec(D), _full_spec((D, D)),
                  pl.BlockSpec((NC, BM, DEG_W), lambda i: (0, i, 0))],
        out_specs=_half_spec(),
        out_shape=jax.ShapeDtypeStruct((NC, N, DH), jnp.float32),
    )(x, w, degp)


def _mid_call(p, hp, degp, w, a, bc):
    return pl.pallas_call(
        _mid_body,
        grid=(N // BM,),
        in_specs=[_half_spec(), _half_spec(),
                  pl.BlockSpec((NC, BM, DEG_W), lambda i: (0, i, 0)),
                  _full_spec((D, D)), _full_spec((1, D)), _full_spec((1, D))],
        out_specs=[_row_spec(D), _half_spec()],
        out_shape=[jax.ShapeDtypeStruct((N, D), jnp.float32),
                   jax.ShapeDtypeStruct((NC, N, DH), jnp.float32)],
    )(p, hp, degp, w, a, bc)


def _fin_call(q, hp2, degp, x0, x1, wfc, a, bc, bfc):
    return pl.pallas_call(
        _fin_body,
        grid=(N // BM,),
        in_specs=[_half_spec(), _half_spec(),
                  pl.BlockSpec((NC, BM, DEG_W), lambda i: (0, i, 0)),
                  _row_spec(D), _row_spec(D), _full_spec((3 * D, D_OUT)),
                  _full_spec((1, D)), _full_spec((1, D)),
                  _full_spec((1, D_OUT))],
        out_specs=_row_spec(D_OUT),
        out_shape=jax.ShapeDtypeStruct((N, D_OUT), jnp.float32),
    )(q, hp2, degp, x0, x1, wfc, a, bc, bfc)


def kernel(x, edge_index, W1, b1, g1, be1, W2, b2, g2, be2, Wfc, bfc):
    sbn = 1.0 / jnp.sqrt(jnp.float32(1.0 + 1e-5))
    a1 = (g1 * sbn).reshape(1, D)
    bc1 = (b1 * g1 * sbn + be1).reshape(1, D)
    a2 = (g2 * sbn).reshape(1, D)
    bc2 = (b2 * g2 * sbn + be2).reshape(1, D)

    er = edge_index.reshape(2 * E)
    degp = _deg_call(er)
    h1p = _pre_call(x, W1, degp)
    p1 = _agg_call(h1p, er)
    x1, h2p = _mid_call(p1, h1p, degp, W2, a1, bc1)
    p2 = _agg_call(h2p, er)
    out = _fin_call(p2, h2p, degp, x, x1, Wfc, a2, bc2, bfc.reshape(1, D_OUT))
    return out
